# baseline (speedup 1.0000x reference)
import functools

import jax
import jax.numpy as jnp
from jax.experimental import pallas as pl
from jax.experimental.pallas import tpu as pltpu

_S = float(1.0 / (1.0 + 1e-5) ** 0.5)
_PN = (2048, 512, 512, 128)
_CID = (0, 2048, 2560, 3072)
_K = 32
_TN = (256, 256, 256, 128)
_INTERPRET = False


def _pad2(a, r, c):
    out = jnp.zeros((r, c), a.dtype)
    return out.at[: a.shape[0], : a.shape[1]].set(a)


def _dot(a, b):
    return jax.lax.dot(a, b, preferred_element_type=jnp.float32)


def _relu(x):
    return jnp.maximum(x, 0.0)


def _bf(x):
    return x.astype(jnp.bfloat16)


def _onehot(idx_row, n):
    io = jax.lax.broadcasted_iota(jnp.int32, (128, idx_row.shape[1]), 0)
    return (io == idx_row).astype(jnp.float32)


def _maxk(h, tn):
    m = h[:, :tn]
    for k in range(1, _K):
        m = jnp.maximum(m, h[:, k * tn:(k + 1) * tn])
    return m


def _axis_body(TN, has_g, has_tbl,
               tbl_ref, nbf_ref, dif_ref,
               w1, b1, w2, b2, w3, b3,
               w4a, w4b, b4, w5, b5, w6, b6,
               w7a, w7b, b7, w8, b8, w9, b9,
               f1w, f1b, f2w, f2b, f3w, f3b,
               *outs):
    o_std = outs[0]
    o_lc = outs[1]
    i = 2
    o_g = None
    o_tbl = None
    if has_g:
        o_g = outs[i]; i += 1
    if has_tbl:
        o_tbl = outs[i]; i += 1
    gsc = outs[i]

    b = pl.program_id(0)
    t = pl.program_id(1)
    NKT = _K * TN

    tblp = tbl_ref[0]
    nbf = nbf_ref[0, 0]
    dif = dif_ref[0]

    cur = _dot(tblp, _onehot(dif, TN))

    @pl.when(t == 0)
    def _():
        gsc[...] = cur[:, :128]

    tblc = gsc[...]
    x0 = _dot(tblc, _onehot(nbf, NKT))

    x0b = _bf(x0)
    h = _relu(_dot(w1[...], x0b) + b1[...])
    h = _relu(_dot(w2[...], _bf(h)) + b2[...])
    l1 = _relu(_dot(w3[...], _bf(h)) + b3[...])
    h = _relu(_dot(w4a[...], x0b) + _dot(w4b[...], _bf(l1)) + b4[...])
    h = _relu(_dot(w5[...], _bf(h)) + b5[...])
    l2 = _relu(_dot(w6[...], _bf(h)) + b6[...])
    h = _relu(_dot(w7a[...], x0b) + _dot(w7b[...], _bf(l2)) + b7[...])
    h = _relu(_dot(w8[...], _bf(h)) + b8[...])
    l3 = _relu(_dot(w9[...], _bf(h)) + b9[...])

    m = _maxk(l3, TN)
    xm = _relu(_dot(f1w[...], m) + f1b[...])
    xm = _relu(_dot(f2w[...], xm) + f2b[...])
    al = _dot(f3w[...], xm) + f3b[...]

    a10, a11, a12 = al[0:1], al[1:2], al[2:3]
    a20, a21, a22 = al[3:4], al[4:5], al[5:6]
    a1n = jnp.sqrt(a10 * a10 + a11 * a11 + a12 * a12) + 1e-9
    kk = (a10 * a20 + a11 * a21 + a12 * a22) / (a1n * a1n)
    b20 = a20 - kk * a10
    b21 = a21 - kk * a11
    b22 = a22 - kk * a12
    bn = jnp.sqrt(b20 * b20 + b21 * b21 + b22 * b22) + 1e-9
    ax0, ax1, ax2 = b20 / bn, b21 / bn, b22 / bn
    az0, az1, az2 = a10 / a1n, a11 / a1n, a12 / a1n
    ay0 = az1 * ax2 - az2 * ax1
    ay1 = az2 * ax0 - az0 * ax2
    ay2 = az0 * ax1 - az1 * ax0

    cur0, cur1, cur2 = cur[0:1], cur[1:2], cur[2:3]
    s0 = jnp.zeros((1, TN), jnp.float32)
    q0 = jnp.zeros((1, TN), jnp.float32)
    s1 = jnp.zeros((1, TN), jnp.float32)
    q1 = jnp.zeros((1, TN), jnp.float32)
    lcx_p, lcy_p, lcz_p = [], [], []
    g0_p, g1_p, g2_p = [], [], []
    for k in range(_K):
        sl = slice(k * TN, (k + 1) * TN)
        g0 = x0[0:1, sl] - cur0
        g1 = x0[1:2, sl] - cur1
        g2 = x0[2:3, sl] - cur2
        lcx = g0 * ax0 + g1 * ax1 + g2 * ax2
        lcy = g0 * ay0 + g1 * ay1 + g2 * ay2
        lcz = g0 * az0 + g1 * az1 + g2 * az2
        s0 += lcx
        q0 += lcx * lcx
        s1 += lcy
        q1 += lcy * lcy
        lcx_p.append(lcx[:, :128])
        lcy_p.append(lcy[:, :128])
        lcz_p.append(lcz[:, :128])
        if has_g:
            g0_p.append(g0[:, :128])
            g1_p.append(g1[:, :128])
            g2_p.append(g2[:, :128])

    v0 = (q0 - s0 * s0 * (1.0 / _K)) * (1.0 / (_K - 1))
    v1 = (q1 - s1 * s1 * (1.0 / _K)) * (1.0 / (_K - 1))
    tot = jnp.sum(jnp.sqrt(jnp.maximum(v0, 0.0)) + jnp.sqrt(jnp.maximum(v1, 0.0)),
                  keepdims=True)

    first = jnp.logical_and(b == 0, t == 0)

    @pl.when(first)
    def _():
        o_std[...] = tot

    @pl.when(jnp.logical_not(first))
    def _():
        o_std[...] = o_std[...] + tot

    z5 = jnp.zeros((5, _K * 128), jnp.float32)
    lcf = jnp.concatenate(
        [jnp.concatenate(lcx_p, axis=1),
         jnp.concatenate(lcy_p, axis=1),
         jnp.concatenate(lcz_p, axis=1), z5], axis=0)

    @pl.when(t == 0)
    def _():
        o_lc[0] = lcf

    if has_g:
        gf = jnp.concatenate(
            [jnp.concatenate(g0_p, axis=1),
             jnp.concatenate(g1_p, axis=1),
             jnp.concatenate(g2_p, axis=1), z5], axis=0)

        @pl.when(t == 0)
        def _():
            o_g[0] = gf

    if has_tbl:
        @pl.when(t == 0)
        def _():
            o_tbl[0] = gsc[...]


def _const_spec(shape):
    n = len(shape)
    return pl.BlockSpec(shape, lambda b, t, _n=n: (0,) * _n)


def _run_axis_level(L, B, tblp, nbf, dif, aw):
    TN = _TN[L]
    pn = _PN[L]
    nt = pn // TN
    has_g = (L == 0)
    has_tbl = (L < 3)

    out_shapes = [jax.ShapeDtypeStruct((1, 1), jnp.float32),
                  jax.ShapeDtypeStruct((B, 8, _K * 128), jnp.float32)]
    out_specs = [pl.BlockSpec((1, 1), lambda b, t: (0, 0)),
                 pl.BlockSpec((1, 8, _K * 128), lambda b, t: (b, 0, 0))]
    if has_g:
        out_shapes.append(jax.ShapeDtypeStruct((B, 8, _K * 128), jnp.float32))
        out_specs.append(pl.BlockSpec((1, 8, _K * 128), lambda b, t: (b, 0, 0)))
    if has_tbl:
        out_shapes.append(jax.ShapeDtypeStruct((B, 8, 128), jnp.float32))
        out_specs.append(pl.BlockSpec((1, 8, 128), lambda b, t: (b, 0, 0)))

    in_specs = [pl.BlockSpec((1, 8, 128), lambda b, t: (b, 0, 0)),
                pl.BlockSpec((1, 1, 1, _K * TN), lambda b, t: (b, t, 0, 0)),
                pl.BlockSpec((1, 1, TN), lambda b, t: (b, 0, t))]
    in_specs += [_const_spec(w.shape) for w in aw]

    fn = pl.pallas_call(
        functools.partial(_axis_body, TN, has_g, has_tbl),
        grid=(B, nt),
        in_specs=in_specs,
        out_specs=out_specs,
        out_shape=out_shapes,
        scratch_shapes=[pltpu.VMEM((8, 128), jnp.float32)],
        interpret=_INTERPRET,
    )
    return fn(tblp, nbf, dif, *aw)


def _head_body(lc0, lc1, lc2, lc3, g0r, G0r, xyzr, dir_, nbr,
               w0a, w0b, b0, w02a, w02b, b02,
               w1a, w1b, b1, w12a, w12b, b12,
               w2a, w2b, b2,
               wm1a, wm1b, bm1, wm2, bm2,
               wf1, bf1, wf2, bf2, wf3, bf3,
               o_ref):
    N = _K * 128
    l0 = lc0[0]
    l1 = lc1[0]
    l2 = lc2[0]
    l3 = lc3[0]
    g0 = g0r[0]
    G0 = G0r[0]
    xyzp = xyzr[0]
    di = dir_[0]
    nb = nbr[0]

    H = _dot(G0, _onehot(di[0:1], 128))
    H = _dot(H, _onehot(di[1:2], 128))
    H = _dot(H, _onehot(di[2:3], 128))
    xyz3 = _dot(H, _onehot(di[3:4], 128))

    oh0 = _bf(_onehot(nb[0:1], N))
    grp = _dot(_bf(xyzp), oh0)
    corr = _dot(w0b[...], _bf(G0))
    corr = jnp.concatenate([corr] * _K, axis=1)
    h = _relu(_dot(w0a[...], _bf(l0)) + _dot(w0b[...], _bf(grp)) - corr
              + b0[...])
    P = _maxk(h, 128)

    h = _relu(_dot(w02a[...], _bf(l0)) + _dot(w02b[...], _bf(_dot(_bf(P), oh0)))
              + b02[...])
    P = _maxk(h, 128)

    gat = _dot(_bf(P), _bf(_onehot(nb[1:2], N)))
    h = _relu(_dot(w1a[...], _bf(l1)) + _dot(w1b[...], _bf(gat)) + b1[...])
    P = _maxk(h, 128)

    gat = _dot(_bf(P), _bf(_onehot(nb[2:3], N)))
    h = _relu(_dot(w12a[...], _bf(l2)) + _dot(w12b[...], _bf(gat)) + b12[...])
    P = _maxk(h, 128)

    gat = _dot(_bf(P), _bf(_onehot(nb[3:4], N)))
    h = _relu(_dot(w2a[...], _bf(l3)) + _dot(w2b[...], _bf(gat)) + b2[...])
    P = _maxk(h, 128)

    h = _relu(_dot(wm1a[...], _bf(xyz3)) + _dot(wm1b[...], _bf(P)) + bm1[...])
    h = _relu(_dot(wm2[...], _bf(h)) + bm2[...])
    v = jnp.max(h, axis=1, keepdims=True)

    x = _relu(_dot(wf1[...], v) + bf1[...])
    x = _relu(_dot(wf2[...], x) + bf2[...])
    z = _dot(wf3[...], x) + bf3[...]
    mz = jnp.max(z, axis=0, keepdims=True)
    e = jnp.exp(z - mz)
    se = jnp.sum(e, axis=0, keepdims=True)
    o_ref[0] = z - mz - jnp.log(se)


def _run_head(B, lcs, g0, G0, xyzp, di128, nbf128, hw):
    data = [lcs[0], lcs[1], lcs[2], lcs[3], g0, G0, xyzp, di128, nbf128]
    in_specs = [pl.BlockSpec((1, 8, _K * 128), lambda b: (b, 0, 0))] * 5
    in_specs += [pl.BlockSpec((1, 8, 128), lambda b: (b, 0, 0))] * 2
    in_specs += [pl.BlockSpec((1, 4, 128), lambda b: (b, 0, 0)),
                 pl.BlockSpec((1, 4, _K * 128), lambda b: (b, 0, 0))]
    in_specs += [pl.BlockSpec(w.shape, lambda b, _n=len(w.shape): (0,) * _n)
                 for w in hw]
    fn = pl.pallas_call(
        _head_body,
        grid=(B,),
        in_specs=in_specs,
        out_specs=pl.BlockSpec((1, 40, 1), lambda b: (b, 0, 0)),
        out_shape=jax.ShapeDtypeStruct((B, 40, 1), jnp.float32),
        interpret=_INTERPRET,
    )
    return fn(*data, *hw)


def _prep_axis_weights(ap):
    s = _S
    (W1, c1), (W2, c2), (W3, c3) = ap['sa1']
    (W4, c4), (W5, c5), (W6, c6) = ap['sa2']
    (W7, c7), (W8, c8), (W9, c9) = ap['sa3']
    f1, f1b = ap['fc1']
    f2, f2b = ap['fc2']
    f3, f3b = ap['fc3']
    bb = lambda v, r: _pad2((v * s).reshape(-1, 1), r, 1)
    cw = lambda w: _bf(jnp.asarray(w, jnp.float32))
    out = [
        cw(_pad2(W1 * s, 8, 8)), bb(c1, 8),
        cw(_pad2(W2 * s, 16, 8)), bb(c2, 16),
        cw(W3 * s), bb(c3, 16),
        cw(_pad2(W4[:, :3] * s, 16, 8)), cw(W4[:, 3:] * s), bb(c4, 16),
        cw(W5 * s), bb(c5, 16),
        cw(W6 * s), bb(c6, 32),
        cw(_pad2(W7[:, :3] * s, 32, 8)), cw(W7[:, 3:] * s), bb(c7, 32),
        cw(W8 * s), bb(c8, 32),
        cw(W9 * s), bb(c9, 64),
        f1.T * s, bb(f1b, 32),
        f2.T * s, bb(f2b, 32),
        _pad2(f3.T, 8, 32), _pad2(f3b.reshape(-1, 1), 8, 1),
    ]
    return [jnp.asarray(w) for w in out]


def _prep_head_weights(p):
    s = _S

    def split(lin, lc_ch, r):
        W, b = lin
        Wt = W.T * s
        return [_bf(_pad2(Wt[:, :lc_ch], r, 8)), _bf(Wt[:, lc_ch:]),
                _pad2((b * s).reshape(-1, 1), r, 1)]

    out = []
    out += split(p['sa0'], 3, 32)
    out[1] = _bf(_pad2(out[1].astype(jnp.float32), 32, 8))
    out += split(p['sa02'], 3, 32)
    out += split(p['sa1'], 3, 128)
    out += split(p['sa12'], 3, 128)
    out += split(p['sa2'], 3, 256)
    m1, m2 = p['merge']
    out += split((m1[0], m1[1]), 3, 256)
    out += [_bf(m2[0].T * s), _pad2((m2[1] * s).reshape(-1, 1), 512, 1)]
    f1, f2, f3 = p['fc1'], p['fc2'], p['fc3']
    out += [f1[0].T * s, _pad2((f1[1] * s).reshape(-1, 1), 256, 1)]
    out += [f2[0].T * s, _pad2((f2[1] * s).reshape(-1, 1), 128, 1)]
    out += [f3[0].T, _pad2(f3[1].reshape(-1, 1), 40, 1)]
    return [jnp.asarray(w) for w in out]


def kernel(xyz, neighbors, data_idxes, params):
    B = xyz.shape[0]
    aw = _prep_axis_weights(params['axis'])
    hw = _prep_head_weights(params)

    xyzp = jnp.zeros((B, 8, 128), jnp.float32).at[:, :3, :].set(
        jnp.transpose(xyz[:, :128, :], (0, 2, 1)))

    nbf, dif = [], []
    for L in range(4):
        pn, TN, cid = _PN[L], _TN[L], _CID[L]
        nt = pn // TN
        nb = neighbors[:, cid:cid + pn, :]
        nbf.append(nb.transpose(0, 2, 1).reshape(B, _K, nt, TN)
                   .transpose(0, 2, 1, 3).reshape(B, nt, 1, _K * TN))
        dif.append(data_idxes[:, cid:cid + pn].reshape(B, 1, pn))

    di128 = jnp.stack([data_idxes[:, _CID[L]:_CID[L] + 128] for L in range(4)],
                      axis=1)
    nbf128 = jnp.stack(
        [neighbors[:, _CID[L]:_CID[L] + 128, :].transpose(0, 2, 1)
         .reshape(B, _K * 128) for L in range(4)], axis=1)

    stds, lcs = [], []
    g0 = None
    G0 = None
    tbl = xyzp
    for L in range(4):
        outs = _run_axis_level(L, B, tbl, nbf[L], dif[L], aw)
        stds.append(outs[0])
        lcs.append(outs[1])
        i = 2
        if L == 0:
            g0 = outs[i]; i += 1
        if L < 3:
            if L == 0:
                G0 = outs[i]
            tbl = outs[i]

    lc_std = sum(stds[L][0, 0] / (B * _PN[L]) for L in range(4))
    out3 = _run_head(B, lcs, g0, G0, xyzp, di128, nbf128, hw)
    return out3[:, :, 0], jnp.float32(lc_std)


# speedup vs baseline: 1.1945x; 1.1945x over previous
import functools

import jax
import jax.numpy as jnp
from jax.experimental import pallas as pl
from jax.experimental.pallas import tpu as pltpu

_S = float(1.0 / (1.0 + 1e-5) ** 0.5)
_PN = (2048, 512, 512, 128)
_CID = (0, 2048, 2560, 3072)
_K = 32
_TN = (512, 512, 512, 128)
_INTERPRET = False


def _pad2(a, r, c):
    out = jnp.zeros((r, c), a.dtype)
    return out.at[: a.shape[0], : a.shape[1]].set(a)


def _dot(a, b):
    return jax.lax.dot(a, b, preferred_element_type=jnp.float32)


def _relu(x):
    return jnp.maximum(x, 0.0)


def _bf(x):
    return x.astype(jnp.bfloat16)


def _onehot(idx_row, n):
    io = jax.lax.broadcasted_iota(jnp.int32, (128, idx_row.shape[1]), 0)
    return (io == idx_row).astype(jnp.float32)


def _maxk(h, tn):
    w = h.shape[1]
    while w > tn:
        w //= 2
        h = jnp.maximum(h[:, :w], h[:, w:2 * w])
    return h


def _foldsum(v, tn):
    w = v.shape[1]
    while w > tn:
        w //= 2
        v = v[:, :w] + v[:, w:2 * w]
    return v


def _tilek(row):
    return jnp.concatenate([row] * _K, axis=1)


def _axis_body(TN, has_g, has_tbl,
               tbl_ref, nbf_ref, dif_ref,
               w1, b1, w2, b2, w3, b3,
               w4a, w4b, b4, w5, b5, w6, b6,
               w7a, w7b, b7, w8, b8, w9, b9,
               f1w, f1b, f2w, f2b, f3w, f3b,
               *outs):
    o_std = outs[0]
    o_lc = outs[1]
    i = 2
    o_g = None
    o_tbl = None
    if has_g:
        o_g = outs[i]; i += 1
    if has_tbl:
        o_tbl = outs[i]; i += 1
    gsc = outs[i]

    b = pl.program_id(0)
    t = pl.program_id(1)
    NKT = _K * TN

    tblp = tbl_ref[0]
    nbf = nbf_ref[0, 0]
    dif = dif_ref[0]

    cur = _dot(tblp, _onehot(dif, TN))

    @pl.when(t == 0)
    def _():
        gsc[...] = cur[:, :128]

    tblc = gsc[...]
    x0 = _dot(tblc, _onehot(nbf, NKT))

    x0b = _bf(x0)
    h = _relu(_dot(w1[...], x0b) + b1[...])
    h = _relu(_dot(w2[...], _bf(h)) + b2[...])
    l1 = _relu(_dot(w3[...], _bf(h)) + b3[...])
    h = _relu(_dot(w4a[...], x0b) + _dot(w4b[...], _bf(l1)) + b4[...])
    h = _relu(_dot(w5[...], _bf(h)) + b5[...])
    l2 = _relu(_dot(w6[...], _bf(h)) + b6[...])
    h = _relu(_dot(w7a[...], x0b) + _dot(w7b[...], _bf(l2)) + b7[...])
    h = _relu(_dot(w8[...], _bf(h)) + b8[...])
    l3 = _relu(_dot(w9[...], _bf(h)) + b9[...])

    m = _maxk(l3, TN)
    xm = _relu(_dot(f1w[...], m) + f1b[...])
    xm = _relu(_dot(f2w[...], xm) + f2b[...])
    al = _dot(f3w[...], xm) + f3b[...]

    a10, a11, a12 = al[0:1], al[1:2], al[2:3]
    a20, a21, a22 = al[3:4], al[4:5], al[5:6]
    a1n = jnp.sqrt(a10 * a10 + a11 * a11 + a12 * a12) + 1e-9
    kk = (a10 * a20 + a11 * a21 + a12 * a22) / (a1n * a1n)
    b20 = a20 - kk * a10
    b21 = a21 - kk * a11
    b22 = a22 - kk * a12
    bn = jnp.sqrt(b20 * b20 + b21 * b21 + b22 * b22) + 1e-9
    ax0, ax1, ax2 = b20 / bn, b21 / bn, b22 / bn
    az0, az1, az2 = a10 / a1n, a11 / a1n, a12 / a1n
    ay0 = az1 * ax2 - az2 * ax1
    ay1 = az2 * ax0 - az0 * ax2
    ay2 = az0 * ax1 - az1 * ax0

    g0f = x0[0:1] - _tilek(cur[0:1])
    g1f = x0[1:2] - _tilek(cur[1:2])
    g2f = x0[2:3] - _tilek(cur[2:3])
    lcxf = g0f * _tilek(ax0) + g1f * _tilek(ax1) + g2f * _tilek(ax2)
    lcyf = g0f * _tilek(ay0) + g1f * _tilek(ay1) + g2f * _tilek(ay2)
    lczf = g0f * _tilek(az0) + g1f * _tilek(az1) + g2f * _tilek(az2)

    s0 = _foldsum(lcxf, TN)
    q0 = _foldsum(lcxf * lcxf, TN)
    s1 = _foldsum(lcyf, TN)
    q1 = _foldsum(lcyf * lcyf, TN)

    lcx_p = [lcxf[:, k * TN:k * TN + 128] for k in range(_K)]
    lcy_p = [lcyf[:, k * TN:k * TN + 128] for k in range(_K)]
    lcz_p = [lczf[:, k * TN:k * TN + 128] for k in range(_K)]
    if has_g:
        g0_p = [g0f[:, k * TN:k * TN + 128] for k in range(_K)]
        g1_p = [g1f[:, k * TN:k * TN + 128] for k in range(_K)]
        g2_p = [g2f[:, k * TN:k * TN + 128] for k in range(_K)]

    v0 = (q0 - s0 * s0 * (1.0 / _K)) * (1.0 / (_K - 1))
    v1 = (q1 - s1 * s1 * (1.0 / _K)) * (1.0 / (_K - 1))
    tot = jnp.sum(jnp.sqrt(jnp.maximum(v0, 0.0)) + jnp.sqrt(jnp.maximum(v1, 0.0)),
                  keepdims=True)

    first = jnp.logical_and(b == 0, t == 0)

    @pl.when(first)
    def _():
        o_std[...] = tot

    @pl.when(jnp.logical_not(first))
    def _():
        o_std[...] = o_std[...] + tot

    z5 = jnp.zeros((5, _K * 128), jnp.float32)
    lcf = jnp.concatenate(
        [jnp.concatenate(lcx_p, axis=1),
         jnp.concatenate(lcy_p, axis=1),
         jnp.concatenate(lcz_p, axis=1), z5], axis=0)

    @pl.when(t == 0)
    def _():
        o_lc[0] = lcf

    if has_g:
        gf = jnp.concatenate(
            [jnp.concatenate(g0_p, axis=1),
             jnp.concatenate(g1_p, axis=1),
             jnp.concatenate(g2_p, axis=1), z5], axis=0)

        @pl.when(t == 0)
        def _():
            o_g[0] = gf

    if has_tbl:
        @pl.when(t == 0)
        def _():
            o_tbl[0] = gsc[...]


def _const_spec(shape):
    n = len(shape)
    return pl.BlockSpec(shape, lambda b, t, _n=n: (0,) * _n)


def _run_axis_level(L, B, tblp, nbf, dif, aw):
    TN = _TN[L]
    pn = _PN[L]
    nt = pn // TN
    has_g = (L == 0)
    has_tbl = (L < 3)

    out_shapes = [jax.ShapeDtypeStruct((1, 1), jnp.float32),
                  jax.ShapeDtypeStruct((B, 8, _K * 128), jnp.float32)]
    out_specs = [pl.BlockSpec((1, 1), lambda b, t: (0, 0)),
                 pl.BlockSpec((1, 8, _K * 128), lambda b, t: (b, 0, 0))]
    if has_g:
        out_shapes.append(jax.ShapeDtypeStruct((B, 8, _K * 128), jnp.float32))
        out_specs.append(pl.BlockSpec((1, 8, _K * 128), lambda b, t: (b, 0, 0)))
    if has_tbl:
        out_shapes.append(jax.ShapeDtypeStruct((B, 8, 128), jnp.float32))
        out_specs.append(pl.BlockSpec((1, 8, 128), lambda b, t: (b, 0, 0)))

    in_specs = [pl.BlockSpec((1, 8, 128), lambda b, t: (b, 0, 0)),
                pl.BlockSpec((1, 1, 1, _K * TN), lambda b, t: (b, t, 0, 0)),
                pl.BlockSpec((1, 1, TN), lambda b, t: (b, 0, t))]
    in_specs += [_const_spec(w.shape) for w in aw]

    fn = pl.pallas_call(
        functools.partial(_axis_body, TN, has_g, has_tbl),
        grid=(B, nt),
        in_specs=in_specs,
        out_specs=out_specs,
        out_shape=out_shapes,
        scratch_shapes=[pltpu.VMEM((8, 128), jnp.float32)],
        interpret=_INTERPRET,
    )
    return fn(tblp, nbf, dif, *aw)


def _head_body(lc0, lc1, lc2, lc3, g0r, G0r, xyzr, dir_, nbr,
               w0a, w0b, b0, w02a, w02b, b02,
               w1a, w1b, b1, w12a, w12b, b12,
               w2a, w2b, b2,
               wm1a, wm1b, bm1, wm2, bm2,
               wf1, bf1, wf2, bf2, wf3, bf3,
               o_ref):
    N = _K * 128
    l0 = lc0[0]
    l1 = lc1[0]
    l2 = lc2[0]
    l3 = lc3[0]
    g0 = g0r[0]
    G0 = G0r[0]
    xyzp = xyzr[0]
    di = dir_[0]
    nb = nbr[0]

    H = _dot(G0, _onehot(di[0:1], 128))
    H = _dot(H, _onehot(di[1:2], 128))
    H = _dot(H, _onehot(di[2:3], 128))
    xyz3 = _dot(H, _onehot(di[3:4], 128))

    oh0 = _bf(_onehot(nb[0:1], N))
    grp = _dot(_bf(xyzp), oh0)
    corr = _dot(w0b[...], _bf(G0))
    corr = jnp.concatenate([corr] * _K, axis=1)
    h = _relu(_dot(w0a[...], _bf(l0)) + _dot(w0b[...], _bf(grp)) - corr
              + b0[...])
    P = _maxk(h, 128)

    h = _relu(_dot(w02a[...], _bf(l0)) + _dot(w02b[...], _bf(_dot(_bf(P), oh0)))
              + b02[...])
    P = _maxk(h, 128)

    gat = _dot(_bf(P), _bf(_onehot(nb[1:2], N)))
    h = _relu(_dot(w1a[...], _bf(l1)) + _dot(w1b[...], _bf(gat)) + b1[...])
    P = _maxk(h, 128)

    gat = _dot(_bf(P), _bf(_onehot(nb[2:3], N)))
    h = _relu(_dot(w12a[...], _bf(l2)) + _dot(w12b[...], _bf(gat)) + b12[...])
    P = _maxk(h, 128)

    gat = _dot(_bf(P), _bf(_onehot(nb[3:4], N)))
    h = _relu(_dot(w2a[...], _bf(l3)) + _dot(w2b[...], _bf(gat)) + b2[...])
    P = _maxk(h, 128)

    h = _relu(_dot(wm1a[...], _bf(xyz3)) + _dot(wm1b[...], _bf(P)) + bm1[...])
    h = _relu(_dot(wm2[...], _bf(h)) + bm2[...])
    v = jnp.max(h, axis=1, keepdims=True)

    x = _relu(_dot(wf1[...], v) + bf1[...])
    x = _relu(_dot(wf2[...], x) + bf2[...])
    z = _dot(wf3[...], x) + bf3[...]
    mz = jnp.max(z, axis=0, keepdims=True)
    e = jnp.exp(z - mz)
    se = jnp.sum(e, axis=0, keepdims=True)
    o_ref[0] = z - mz - jnp.log(se)


def _run_head(B, lcs, g0, G0, xyzp, di128, nbf128, hw):
    data = [lcs[0], lcs[1], lcs[2], lcs[3], g0, G0, xyzp, di128, nbf128]
    in_specs = [pl.BlockSpec((1, 8, _K * 128), lambda b: (b, 0, 0))] * 5
    in_specs += [pl.BlockSpec((1, 8, 128), lambda b: (b, 0, 0))] * 2
    in_specs += [pl.BlockSpec((1, 4, 128), lambda b: (b, 0, 0)),
                 pl.BlockSpec((1, 4, _K * 128), lambda b: (b, 0, 0))]
    in_specs += [pl.BlockSpec(w.shape, lambda b, _n=len(w.shape): (0,) * _n)
                 for w in hw]
    fn = pl.pallas_call(
        _head_body,
        grid=(B,),
        in_specs=in_specs,
        out_specs=pl.BlockSpec((1, 40, 1), lambda b: (b, 0, 0)),
        out_shape=jax.ShapeDtypeStruct((B, 40, 1), jnp.float32),
        interpret=_INTERPRET,
    )
    return fn(*data, *hw)


def _prep_axis_weights(ap):
    s = _S
    (W1, c1), (W2, c2), (W3, c3) = ap['sa1']
    (W4, c4), (W5, c5), (W6, c6) = ap['sa2']
    (W7, c7), (W8, c8), (W9, c9) = ap['sa3']
    f1, f1b = ap['fc1']
    f2, f2b = ap['fc2']
    f3, f3b = ap['fc3']
    bb = lambda v, r: _pad2((v * s).reshape(-1, 1), r, 1)
    cw = lambda w: _bf(jnp.asarray(w, jnp.float32))
    out = [
        cw(_pad2(W1 * s, 8, 8)), bb(c1, 8),
        cw(_pad2(W2 * s, 16, 8)), bb(c2, 16),
        cw(W3 * s), bb(c3, 16),
        cw(_pad2(W4[:, :3] * s, 16, 8)), cw(W4[:, 3:] * s), bb(c4, 16),
        cw(W5 * s), bb(c5, 16),
        cw(W6 * s), bb(c6, 32),
        cw(_pad2(W7[:, :3] * s, 32, 8)), cw(W7[:, 3:] * s), bb(c7, 32),
        cw(W8 * s), bb(c8, 32),
        cw(W9 * s), bb(c9, 64),
        f1.T * s, bb(f1b, 32),
        f2.T * s, bb(f2b, 32),
        _pad2(f3.T, 8, 32), _pad2(f3b.reshape(-1, 1), 8, 1),
    ]
    return [jnp.asarray(w) for w in out]


def _prep_head_weights(p):
    s = _S

    def split(lin, lc_ch, r):
        W, b = lin
        Wt = W.T * s
        return [_bf(_pad2(Wt[:, :lc_ch], r, 8)), _bf(Wt[:, lc_ch:]),
                _pad2((b * s).reshape(-1, 1), r, 1)]

    out = []
    out += split(p['sa0'], 3, 32)
    out[1] = _bf(_pad2(out[1].astype(jnp.float32), 32, 8))
    out += split(p['sa02'], 3, 32)
    out += split(p['sa1'], 3, 128)
    out += split(p['sa12'], 3, 128)
    out += split(p['sa2'], 3, 256)
    m1, m2 = p['merge']
    out += split((m1[0], m1[1]), 3, 256)
    out += [_bf(m2[0].T * s), _pad2((m2[1] * s).reshape(-1, 1), 512, 1)]
    f1, f2, f3 = p['fc1'], p['fc2'], p['fc3']
    out += [f1[0].T * s, _pad2((f1[1] * s).reshape(-1, 1), 256, 1)]
    out += [f2[0].T * s, _pad2((f2[1] * s).reshape(-1, 1), 128, 1)]
    out += [f3[0].T, _pad2(f3[1].reshape(-1, 1), 40, 1)]
    return [jnp.asarray(w) for w in out]


def kernel(xyz, neighbors, data_idxes, params):
    B = xyz.shape[0]
    aw = _prep_axis_weights(params['axis'])
    hw = _prep_head_weights(params)

    xyzp = jnp.zeros((B, 8, 128), jnp.float32).at[:, :3, :].set(
        jnp.transpose(xyz[:, :128, :], (0, 2, 1)))

    nbf, dif = [], []
    for L in range(4):
        pn, TN, cid = _PN[L], _TN[L], _CID[L]
        nt = pn // TN
        nb = neighbors[:, cid:cid + pn, :]
        nbf.append(nb.transpose(0, 2, 1).reshape(B, _K, nt, TN)
                   .transpose(0, 2, 1, 3).reshape(B, nt, 1, _K * TN))
        dif.append(data_idxes[:, cid:cid + pn].reshape(B, 1, pn))

    di128 = jnp.stack([data_idxes[:, _CID[L]:_CID[L] + 128] for L in range(4)],
                      axis=1)
    nbf128 = jnp.stack(
        [neighbors[:, _CID[L]:_CID[L] + 128, :].transpose(0, 2, 1)
         .reshape(B, _K * 128) for L in range(4)], axis=1)

    stds, lcs = [], []
    g0 = None
    G0 = None
    tbl = xyzp
    for L in range(4):
        outs = _run_axis_level(L, B, tbl, nbf[L], dif[L], aw)
        stds.append(outs[0])
        lcs.append(outs[1])
        i = 2
        if L == 0:
            g0 = outs[i]; i += 1
        if L < 3:
            if L == 0:
                G0 = outs[i]
            tbl = outs[i]

    lc_std = sum(stds[L][0, 0] / (B * _PN[L]) for L in range(4))
    out3 = _run_head(B, lcs, g0, G0, xyzp, di128, nbf128, hw)
    return out3[:, :, 0], jnp.float32(lc_std)


# speedup vs baseline: 1.2252x; 1.0257x over previous
import functools

import jax
import jax.numpy as jnp
from jax.experimental import pallas as pl
from jax.experimental.pallas import tpu as pltpu

_S = float(1.0 / (1.0 + 1e-5) ** 0.5)
_PN = (2048, 512, 512, 128)
_CID = (0, 2048, 2560, 3072)
_K = 32
_TN = (512, 512, 512, 128)
_NB = (1, 2, 2, 4)
_NBH = 2
_INTERPRET = False


def _pad2(a, r, c):
    out = jnp.zeros((r, c), a.dtype)
    return out.at[: a.shape[0], : a.shape[1]].set(a)


def _dot(a, b):
    return jax.lax.dot(a, b, preferred_element_type=jnp.float32)


def _relu(x):
    return jnp.maximum(x, 0.0)


def _bf(x):
    return x.astype(jnp.bfloat16)


def _onehot(idx_row, n):
    io = jax.lax.broadcasted_iota(jnp.int32, (128, idx_row.shape[1]), 0)
    return (io == idx_row).astype(jnp.float32)


def _onehotb(idx_row, n):
    io = jax.lax.broadcasted_iota(jnp.int32, (128, idx_row.shape[1]), 0)
    return (io == idx_row).astype(jnp.bfloat16)


def _hilo(tbl):
    hi = _bf(tbl)
    lo = _bf(tbl - hi.astype(jnp.float32))
    return jnp.concatenate([hi, lo], axis=0)


def _maxk(h, tn):
    w = h.shape[1]
    while w > tn:
        w //= 2
        h = jnp.maximum(h[:, :w], h[:, w:2 * w])
    return h


def _foldsum(v, tn):
    w = v.shape[1]
    while w > tn:
        w //= 2
        v = v[:, :w] + v[:, w:2 * w]
    return v


def _tilek(row):
    return jnp.concatenate([row] * _K, axis=1)


def _axis_body(TN, NB, has_g, has_tbl,
               tbl_ref, nbf_ref, dif_ref,
               w1, b1, w2, b2, w3, b3,
               w4a, w4b, b4, w5, b5, w6, b6,
               w7a, w7b, b7, w8, b8, w9, b9,
               f1w, f1b, f2w, f2b, f3w, f3b,
               *outs):
    o_std = outs[0]
    o_lc = outs[1]
    i = 2
    o_g = None
    o_tbl = None
    if has_g:
        o_g = outs[i]; i += 1
    if has_tbl:
        o_tbl = outs[i]; i += 1
    gsc = outs[i]

    b = pl.program_id(0)
    t = pl.program_id(1)
    NKT = _K * TN

    tots = []
    for j in range(NB):
        tots.append(_axis_one(TN, has_g, has_tbl, j, t,
                              tbl_ref, nbf_ref, dif_ref,
                              w1, b1, w2, b2, w3, b3,
                              w4a, w4b, b4, w5, b5, w6, b6,
                              w7a, w7b, b7, w8, b8, w9, b9,
                              f1w, f1b, f2w, f2b, f3w, f3b,
                              o_lc, o_g, o_tbl, gsc))
    tot = tots[0]
    for v in tots[1:]:
        tot = tot + v

    first = jnp.logical_and(b == 0, t == 0)

    @pl.when(first)
    def _():
        o_std[...] = tot

    @pl.when(jnp.logical_not(first))
    def _():
        o_std[...] = o_std[...] + tot


def _axis_one(TN, has_g, has_tbl, j, t,
              tbl_ref, nbf_ref, dif_ref,
              w1, b1, w2, b2, w3, b3,
              w4a, w4b, b4, w5, b5, w6, b6,
              w7a, w7b, b7, w8, b8, w9, b9,
              f1w, f1b, f2w, f2b, f3w, f3b,
              o_lc, o_g, o_tbl, gsc):
    NKT = _K * TN
    tblp = tbl_ref[j]
    nbf = nbf_ref[j, 0]
    dif = dif_ref[j]

    chl = _dot(_hilo(tblp), _onehotb(dif, TN))
    cur = chl[0:8] + chl[8:16]

    @pl.when(t == 0)
    def _():
        gsc[j] = cur[:, :128]

    tblc = gsc[j]
    xhl = _dot(_hilo(tblc), _onehotb(nbf, NKT))
    x0 = xhl[0:8] + xhl[8:16]

    x0b = _bf(xhl[0:8])
    h = _relu(_dot(w1[...], x0b) + b1[...])
    h = _relu(_dot(w2[...], _bf(h)) + b2[...])
    l1 = _relu(_dot(w3[...], _bf(h)) + b3[...])
    h = _relu(_dot(w4a[...], x0b) + _dot(w4b[...], _bf(l1)) + b4[...])
    h = _relu(_dot(w5[...], _bf(h)) + b5[...])
    l2 = _relu(_dot(w6[...], _bf(h)) + b6[...])
    h = _relu(_dot(w7a[...], x0b) + _dot(w7b[...], _bf(l2)) + b7[...])
    h = _relu(_dot(w8[...], _bf(h)) + b8[...])
    l3 = _relu(_dot(w9[...], _bf(h)) + b9[...])

    m = _maxk(l3, TN)
    xm = _relu(_dot(f1w[...], m) + f1b[...])
    xm = _relu(_dot(f2w[...], xm) + f2b[...])
    al = _dot(f3w[...], xm) + f3b[...]

    a10, a11, a12 = al[0:1], al[1:2], al[2:3]
    a20, a21, a22 = al[3:4], al[4:5], al[5:6]
    a1n = jnp.sqrt(a10 * a10 + a11 * a11 + a12 * a12) + 1e-9
    kk = (a10 * a20 + a11 * a21 + a12 * a22) / (a1n * a1n)
    b20 = a20 - kk * a10
    b21 = a21 - kk * a11
    b22 = a22 - kk * a12
    bn = jnp.sqrt(b20 * b20 + b21 * b21 + b22 * b22) + 1e-9
    ax0, ax1, ax2 = b20 / bn, b21 / bn, b22 / bn
    az0, az1, az2 = a10 / a1n, a11 / a1n, a12 / a1n
    ay0 = az1 * ax2 - az2 * ax1
    ay1 = az2 * ax0 - az0 * ax2
    ay2 = az0 * ax1 - az1 * ax0

    g0f = x0[0:1] - _tilek(cur[0:1])
    g1f = x0[1:2] - _tilek(cur[1:2])
    g2f = x0[2:3] - _tilek(cur[2:3])
    lcxf = g0f * _tilek(ax0) + g1f * _tilek(ax1) + g2f * _tilek(ax2)
    lcyf = g0f * _tilek(ay0) + g1f * _tilek(ay1) + g2f * _tilek(ay2)
    lczf = g0f * _tilek(az0) + g1f * _tilek(az1) + g2f * _tilek(az2)

    s0 = _foldsum(lcxf, TN)
    q0 = _foldsum(lcxf * lcxf, TN)
    s1 = _foldsum(lcyf, TN)
    q1 = _foldsum(lcyf * lcyf, TN)

    lcx_p = [lcxf[:, k * TN:k * TN + 128] for k in range(_K)]
    lcy_p = [lcyf[:, k * TN:k * TN + 128] for k in range(_K)]
    lcz_p = [lczf[:, k * TN:k * TN + 128] for k in range(_K)]
    if has_g:
        g0_p = [g0f[:, k * TN:k * TN + 128] for k in range(_K)]
        g1_p = [g1f[:, k * TN:k * TN + 128] for k in range(_K)]
        g2_p = [g2f[:, k * TN:k * TN + 128] for k in range(_K)]

    v0 = (q0 - s0 * s0 * (1.0 / _K)) * (1.0 / (_K - 1))
    v1 = (q1 - s1 * s1 * (1.0 / _K)) * (1.0 / (_K - 1))
    tot = jnp.sum(jnp.sqrt(jnp.maximum(v0, 0.0)) + jnp.sqrt(jnp.maximum(v1, 0.0)),
                  keepdims=True)

    z5 = jnp.zeros((5, _K * 128), jnp.float32)
    lcf = jnp.concatenate(
        [jnp.concatenate(lcx_p, axis=1),
         jnp.concatenate(lcy_p, axis=1),
         jnp.concatenate(lcz_p, axis=1), z5], axis=0)

    @pl.when(t == 0)
    def _():
        o_lc[j] = lcf

    if has_g:
        gf = jnp.concatenate(
            [jnp.concatenate(g0_p, axis=1),
             jnp.concatenate(g1_p, axis=1),
             jnp.concatenate(g2_p, axis=1), z5], axis=0)

        @pl.when(t == 0)
        def _():
            o_g[j] = gf

    if has_tbl:
        @pl.when(t == 0)
        def _():
            o_tbl[j] = gsc[j]

    return tot


def _const_spec(shape):
    n = len(shape)
    return pl.BlockSpec(shape, lambda b, t, _n=n: (0,) * _n)


def _run_axis_level(L, B, tblp, nbf, dif, aw):
    TN = _TN[L]
    NB = _NB[L]
    pn = _PN[L]
    nt = pn // TN
    has_g = (L == 0)
    has_tbl = (L < 3)

    out_shapes = [jax.ShapeDtypeStruct((1, 1), jnp.float32),
                  jax.ShapeDtypeStruct((B, 8, _K * 128), jnp.float32)]
    out_specs = [pl.BlockSpec((1, 1), lambda b, t: (0, 0)),
                 pl.BlockSpec((NB, 8, _K * 128), lambda b, t: (b, 0, 0))]
    if has_g:
        out_shapes.append(jax.ShapeDtypeStruct((B, 8, _K * 128), jnp.float32))
        out_specs.append(pl.BlockSpec((NB, 8, _K * 128), lambda b, t: (b, 0, 0)))
    if has_tbl:
        out_shapes.append(jax.ShapeDtypeStruct((B, 8, 128), jnp.float32))
        out_specs.append(pl.BlockSpec((NB, 8, 128), lambda b, t: (b, 0, 0)))

    in_specs = [pl.BlockSpec((NB, 8, 128), lambda b, t: (b, 0, 0)),
                pl.BlockSpec((NB, 1, 1, _K * TN), lambda b, t: (b, t, 0, 0)),
                pl.BlockSpec((NB, 1, TN), lambda b, t: (b, 0, t))]
    in_specs += [_const_spec(w.shape) for w in aw]

    fn = pl.pallas_call(
        functools.partial(_axis_body, TN, NB, has_g, has_tbl),
        grid=(B // NB, nt),
        in_specs=in_specs,
        out_specs=out_specs,
        out_shape=out_shapes,
        scratch_shapes=[pltpu.VMEM((NB, 8, 128), jnp.float32)],
        interpret=_INTERPRET,
    )
    return fn(tblp, nbf, dif, *aw)


def _head_body(lc0, lc1, lc2, lc3, g0r, G0r, xyzr, dir_, nbr,
               w0a, w0b, b0, w02a, w02b, b02,
               w1a, w1b, b1, w12a, w12b, b12,
               w2a, w2b, b2,
               wm1a, wm1b, bm1, wm2, bm2,
               wf1, bf1, wf2, bf2, wf3, bf3,
               o_ref):
    for j in range(_NBH):
        _head_one(j, lc0, lc1, lc2, lc3, g0r, G0r, xyzr, dir_, nbr,
                  w0a, w0b, b0, w02a, w02b, b02,
                  w1a, w1b, b1, w12a, w12b, b12,
                  w2a, w2b, b2,
                  wm1a, wm1b, bm1, wm2, bm2,
                  wf1, bf1, wf2, bf2, wf3, bf3,
                  o_ref)


def _head_one(j, lc0, lc1, lc2, lc3, g0r, G0r, xyzr, dir_, nbr,
              w0a, w0b, b0, w02a, w02b, b02,
              w1a, w1b, b1, w12a, w12b, b12,
              w2a, w2b, b2,
              wm1a, wm1b, bm1, wm2, bm2,
              wf1, bf1, wf2, bf2, wf3, bf3,
              o_ref):
    N = _K * 128
    l0 = lc0[j]
    l1 = lc1[j]
    l2 = lc2[j]
    l3 = lc3[j]
    g0 = g0r[j]
    G0 = G0r[j]
    xyzp = xyzr[j]
    di = dir_[j]
    nb = nbr[j]

    H = _dot(G0, _onehot(di[0:1], 128))
    H = _dot(H, _onehot(di[1:2], 128))
    H = _dot(H, _onehot(di[2:3], 128))
    xyz3 = _dot(H, _onehot(di[3:4], 128))

    oh0 = _onehotb(nb[0:1], N)
    grp = _dot(_bf(xyzp), oh0)
    corr = _dot(w0b[...], _bf(G0))
    corr = jnp.concatenate([corr] * _K, axis=1)
    h = _relu(_dot(w0a[...], _bf(l0)) + _dot(w0b[...], _bf(grp)) - corr
              + b0[...])
    P = _maxk(h, 128)

    h = _relu(_dot(w02a[...], _bf(l0)) + _dot(w02b[...], _bf(_dot(_bf(P), oh0)))
              + b02[...])
    P = _maxk(h, 128)

    gat = _dot(_bf(P), _onehotb(nb[1:2], N))
    h = _relu(_dot(w1a[...], _bf(l1)) + _dot(w1b[...], _bf(gat)) + b1[...])
    P = _maxk(h, 128)

    gat = _dot(_bf(P), _onehotb(nb[2:3], N))
    h = _relu(_dot(w12a[...], _bf(l2)) + _dot(w12b[...], _bf(gat)) + b12[...])
    P = _maxk(h, 128)

    gat = _dot(_bf(P), _onehotb(nb[3:4], N))
    h = _relu(_dot(w2a[...], _bf(l3)) + _dot(w2b[...], _bf(gat)) + b2[...])
    P = _maxk(h, 128)

    h = _relu(_dot(wm1a[...], _bf(xyz3)) + _dot(wm1b[...], _bf(P)) + bm1[...])
    h = _relu(_dot(wm2[...], _bf(h)) + bm2[...])
    v = jnp.max(h, axis=1, keepdims=True)

    x = _relu(_dot(wf1[...], v) + bf1[...])
    x = _relu(_dot(wf2[...], x) + bf2[...])
    z = _dot(wf3[...], x) + bf3[...]
    mz = jnp.max(z, axis=0, keepdims=True)
    e = jnp.exp(z - mz)
    se = jnp.sum(e, axis=0, keepdims=True)
    o_ref[j] = z - mz - jnp.log(se)


def _run_head(B, lcs, g0, G0, xyzp, di128, nbf128, hw):
    data = [lcs[0], lcs[1], lcs[2], lcs[3], g0, G0, xyzp, di128, nbf128]
    NB = _NBH
    in_specs = [pl.BlockSpec((NB, 8, _K * 128), lambda b: (b, 0, 0))] * 5
    in_specs += [pl.BlockSpec((NB, 8, 128), lambda b: (b, 0, 0))] * 2
    in_specs += [pl.BlockSpec((NB, 4, 128), lambda b: (b, 0, 0)),
                 pl.BlockSpec((NB, 4, _K * 128), lambda b: (b, 0, 0))]
    in_specs += [pl.BlockSpec(w.shape, lambda b, _n=len(w.shape): (0,) * _n)
                 for w in hw]
    fn = pl.pallas_call(
        _head_body,
        grid=(B // NB,),
        in_specs=in_specs,
        out_specs=pl.BlockSpec((NB, 40, 1), lambda b: (b, 0, 0)),
        out_shape=jax.ShapeDtypeStruct((B, 40, 1), jnp.float32),
        interpret=_INTERPRET,
    )
    return fn(*data, *hw)


def _prep_axis_weights(ap):
    s = _S
    (W1, c1), (W2, c2), (W3, c3) = ap['sa1']
    (W4, c4), (W5, c5), (W6, c6) = ap['sa2']
    (W7, c7), (W8, c8), (W9, c9) = ap['sa3']
    f1, f1b = ap['fc1']
    f2, f2b = ap['fc2']
    f3, f3b = ap['fc3']
    bb = lambda v, r: _pad2((v * s).reshape(-1, 1), r, 1)
    cw = lambda w: _bf(jnp.asarray(w, jnp.float32))
    out = [
        cw(_pad2(W1 * s, 8, 8)), bb(c1, 8),
        cw(_pad2(W2 * s, 16, 8)), bb(c2, 16),
        cw(W3 * s), bb(c3, 16),
        cw(_pad2(W4[:, :3] * s, 16, 8)), cw(W4[:, 3:] * s), bb(c4, 16),
        cw(W5 * s), bb(c5, 16),
        cw(W6 * s), bb(c6, 32),
        cw(_pad2(W7[:, :3] * s, 32, 8)), cw(W7[:, 3:] * s), bb(c7, 32),
        cw(W8 * s), bb(c8, 32),
        cw(W9 * s), bb(c9, 64),
        f1.T * s, bb(f1b, 32),
        f2.T * s, bb(f2b, 32),
        _pad2(f3.T, 8, 32), _pad2(f3b.reshape(-1, 1), 8, 1),
    ]
    return [jnp.asarray(w) for w in out]


def _prep_head_weights(p):
    s = _S

    def split(lin, lc_ch, r):
        W, b = lin
        Wt = W.T * s
        return [_bf(_pad2(Wt[:, :lc_ch], r, 8)), _bf(Wt[:, lc_ch:]),
                _pad2((b * s).reshape(-1, 1), r, 1)]

    out = []
    out += split(p['sa0'], 3, 32)
    out[1] = _bf(_pad2(out[1].astype(jnp.float32), 32, 8))
    out += split(p['sa02'], 3, 32)
    out += split(p['sa1'], 3, 128)
    out += split(p['sa12'], 3, 128)
    out += split(p['sa2'], 3, 256)
    m1, m2 = p['merge']
    out += split((m1[0], m1[1]), 3, 256)
    out += [_bf(m2[0].T * s), _pad2((m2[1] * s).reshape(-1, 1), 512, 1)]
    f1, f2, f3 = p['fc1'], p['fc2'], p['fc3']
    out += [f1[0].T * s, _pad2((f1[1] * s).reshape(-1, 1), 256, 1)]
    out += [f2[0].T * s, _pad2((f2[1] * s).reshape(-1, 1), 128, 1)]
    out += [f3[0].T, _pad2(f3[1].reshape(-1, 1), 40, 1)]
    return [jnp.asarray(w) for w in out]


def kernel(xyz, neighbors, data_idxes, params):
    B = xyz.shape[0]
    aw = _prep_axis_weights(params['axis'])
    hw = _prep_head_weights(params)

    xyzp = jnp.zeros((B, 8, 128), jnp.float32).at[:, :3, :].set(
        jnp.transpose(xyz[:, :128, :], (0, 2, 1)))

    nbf, dif = [], []
    for L in range(4):
        pn, TN, cid = _PN[L], _TN[L], _CID[L]
        nt = pn // TN
        nb = neighbors[:, cid:cid + pn, :]
        nbf.append(nb.transpose(0, 2, 1).reshape(B, _K, nt, TN)
                   .transpose(0, 2, 1, 3).reshape(B, nt, 1, _K * TN))
        dif.append(data_idxes[:, cid:cid + pn].reshape(B, 1, pn))

    di128 = jnp.stack([data_idxes[:, _CID[L]:_CID[L] + 128] for L in range(4)],
                      axis=1)
    nbf128 = jnp.stack(
        [neighbors[:, _CID[L]:_CID[L] + 128, :].transpose(0, 2, 1)
         .reshape(B, _K * 128) for L in range(4)], axis=1)

    stds, lcs = [], []
    g0 = None
    G0 = None
    tbl = xyzp
    for L in range(4):
        outs = _run_axis_level(L, B, tbl, nbf[L], dif[L], aw)
        stds.append(outs[0])
        lcs.append(outs[1])
        i = 2
        if L == 0:
            g0 = outs[i]; i += 1
        if L < 3:
            if L == 0:
                G0 = outs[i]
            tbl = outs[i]

    lc_std = sum(stds[L][0, 0] / (B * _PN[L]) for L in range(4))
    out3 = _run_head(B, lcs, g0, G0, xyzp, di128, nbf128, hw)
    return out3[:, :, 0], jnp.float32(lc_std)


# speedup vs baseline: 1.2598x; 1.0282x over previous
import functools

import jax
import jax.numpy as jnp
from jax import lax
from jax.experimental import pallas as pl
from jax.experimental.pallas import tpu as pltpu
from jax.experimental.pallas import tpu_sc as plsc

_S = float(1.0 / (1.0 + 1e-5) ** 0.5)
_PN = (2048, 512, 512, 128)
_CID = (0, 2048, 2560, 3072)
_K = 32
_TN = (512, 512, 512, 128)
_NB = (1, 2, 2, 4)
_NBH = 2
_INTERPRET = False


def _pad2(a, r, c):
    out = jnp.zeros((r, c), a.dtype)
    return out.at[: a.shape[0], : a.shape[1]].set(a)


def _dot(a, b):
    return jax.lax.dot(a, b, preferred_element_type=jnp.float32)


def _relu(x):
    return jnp.maximum(x, 0.0)


def _bf(x):
    return x.astype(jnp.bfloat16)


def _onehot(idx_row, n):
    io = jax.lax.broadcasted_iota(jnp.int32, (128, idx_row.shape[1]), 0)
    return (io == idx_row).astype(jnp.float32)


def _onehotb(idx_row, n):
    io = jax.lax.broadcasted_iota(jnp.int32, (128, idx_row.shape[1]), 0)
    return (io == idx_row).astype(jnp.bfloat16)


def _hilo(tbl):
    hi = _bf(tbl)
    lo = _bf(tbl - hi.astype(jnp.float32))
    return jnp.concatenate([hi, lo], axis=0)


def _maxk(h, tn):
    w = h.shape[1]
    while w > tn:
        w //= 2
        h = jnp.maximum(h[:, :w], h[:, w:2 * w])
    return h


def _foldsum(v, tn):
    w = v.shape[1]
    while w > tn:
        w //= 2
        v = v[:, :w] + v[:, w:2 * w]
    return v


def _tilek(row):
    return jnp.concatenate([row] * _K, axis=1)


def _sc_level(B, srcg, srch, di, h_from_g):
    mesh = plsc.VectorSubcoreMesh(core_axis_name="c", subcore_axis_name="s")

    @functools.partial(
        pl.kernel, mesh=mesh,
        out_type=[jax.ShapeDtypeStruct((B * 128, 128), jnp.float32),
                  jax.ShapeDtypeStruct((B * 128, 128), jnp.float32)],
        scratch_types=[pltpu.VMEM((128,), jnp.int32),
                       pltpu.VMEM((128,), jnp.int32),
                       pltpu.VMEM((128, 128), jnp.float32),
                       pltpu.VMEM((128, 128), jnp.float32),
                       pltpu.SemaphoreType.DMA,
                       pltpu.SemaphoreType.DMA],
    )
    def k(srcg_h, srch_h, di_h, outg, outh, dvi, idxb, rows, rows2, s1, s2):
        wid = lax.axis_index("s") * 2 + lax.axis_index("c")
        base = wid * 128
        pltpu.sync_copy(di_h.at[wid], dvi)
        for i in range(8):
            idxb[pl.ds(i * 16, 16)] = dvi[pl.ds(i * 16, 16)] + base
        pltpu.async_copy(srcg_h.at[idxb], rows, s1).wait()
        pltpu.sync_copy(rows, outg.at[pl.ds(base, 128)])
        hsrc = outg if h_from_g else srch_h
        pltpu.async_copy(hsrc.at[idxb], rows2, s2).wait()
        pltpu.sync_copy(rows2, outh.at[pl.ds(base, 128)])

    return k(srcg, srch, di)


def _sc_tables(B, xyzr, di128):
    g, h = xyzr, xyzr
    tabs = []
    for L in range(4):
        g, h = _sc_level(B, g, h, di128[:, L], h_from_g=(L == 0))
        tabs.append(g)
    z5 = jnp.zeros((B, 5, 128), jnp.float32)

    def _cm(tb):
        t = jnp.transpose(tb.reshape(B, 128, 128)[:, :, :3], (0, 2, 1))
        return jnp.concatenate([t, z5], axis=1)

    return [_cm(t) for t in tabs], _cm(h)


def _axis_body(TN, NB, has_g,
               tbl_ref, tblc_ref, nbf_ref, dif_ref,
               w1, b1, w2, b2, w3, b3,
               w4a, w4b, b4, w5, b5, w6, b6,
               w7a, w7b, b7, w8, b8, w9, b9,
               f1w, f1b, f2w, f2b, f3w, f3b,
               *outs):
    o_std = outs[0]
    o_lc = outs[1]
    o_g = outs[2] if has_g else None

    b = pl.program_id(0)
    t = pl.program_id(1)

    tots = []
    for j in range(NB):
        tots.append(_axis_one(TN, has_g, j, t,
                              tbl_ref, tblc_ref, nbf_ref, dif_ref,
                              w1, b1, w2, b2, w3, b3,
                              w4a, w4b, b4, w5, b5, w6, b6,
                              w7a, w7b, b7, w8, b8, w9, b9,
                              f1w, f1b, f2w, f2b, f3w, f3b,
                              o_lc, o_g))
    tot = tots[0]
    for v in tots[1:]:
        tot = tot + v

    first = jnp.logical_and(b == 0, t == 0)

    @pl.when(first)
    def _():
        o_std[...] = tot

    @pl.when(jnp.logical_not(first))
    def _():
        o_std[...] = o_std[...] + tot


def _axis_one(TN, has_g, j, t,
              tbl_ref, tblc_ref, nbf_ref, dif_ref,
              w1, b1, w2, b2, w3, b3,
              w4a, w4b, b4, w5, b5, w6, b6,
              w7a, w7b, b7, w8, b8, w9, b9,
              f1w, f1b, f2w, f2b, f3w, f3b,
              o_lc, o_g):
    NKT = _K * TN
    tblp = tbl_ref[j]
    tblc = tblc_ref[j]
    nbf = nbf_ref[j, 0]
    dif = dif_ref[j]

    chl = _dot(_hilo(tblp), _onehotb(dif, TN))
    cur = chl[0:8] + chl[8:16]

    xhl = _dot(_hilo(tblc), _onehotb(nbf, NKT))
    x0 = xhl[0:8] + xhl[8:16]

    x0b = _bf(xhl[0:8])
    h = _relu(_dot(w1[...], x0b) + b1[...])
    h = _relu(_dot(w2[...], _bf(h)) + b2[...])
    l1 = _relu(_dot(w3[...], _bf(h)) + b3[...])
    h = _relu(_dot(w4a[...], x0b) + _dot(w4b[...], _bf(l1)) + b4[...])
    h = _relu(_dot(w5[...], _bf(h)) + b5[...])
    l2 = _relu(_dot(w6[...], _bf(h)) + b6[...])
    h = _relu(_dot(w7a[...], x0b) + _dot(w7b[...], _bf(l2)) + b7[...])
    h = _relu(_dot(w8[...], _bf(h)) + b8[...])
    l3 = _relu(_dot(w9[...], _bf(h)) + b9[...])

    m = _maxk(l3, TN)
    xm = _relu(_dot(f1w[...], m) + f1b[...])
    xm = _relu(_dot(f2w[...], xm) + f2b[...])
    al = _dot(f3w[...], xm) + f3b[...]

    a10, a11, a12 = al[0:1], al[1:2], al[2:3]
    a20, a21, a22 = al[3:4], al[4:5], al[5:6]
    a1n = jnp.sqrt(a10 * a10 + a11 * a11 + a12 * a12) + 1e-9
    kk = (a10 * a20 + a11 * a21 + a12 * a22) / (a1n * a1n)
    b20 = a20 - kk * a10
    b21 = a21 - kk * a11
    b22 = a22 - kk * a12
    bn = jnp.sqrt(b20 * b20 + b21 * b21 + b22 * b22) + 1e-9
    ax0, ax1, ax2 = b20 / bn, b21 / bn, b22 / bn
    az0, az1, az2 = a10 / a1n, a11 / a1n, a12 / a1n
    ay0 = az1 * ax2 - az2 * ax1
    ay1 = az2 * ax0 - az0 * ax2
    ay2 = az0 * ax1 - az1 * ax0

    g0f = x0[0:1] - _tilek(cur[0:1])
    g1f = x0[1:2] - _tilek(cur[1:2])
    g2f = x0[2:3] - _tilek(cur[2:3])
    lcxf = g0f * _tilek(ax0) + g1f * _tilek(ax1) + g2f * _tilek(ax2)
    lcyf = g0f * _tilek(ay0) + g1f * _tilek(ay1) + g2f * _tilek(ay2)
    lczf = g0f * _tilek(az0) + g1f * _tilek(az1) + g2f * _tilek(az2)

    s0 = _foldsum(lcxf, TN)
    q0 = _foldsum(lcxf * lcxf, TN)
    s1 = _foldsum(lcyf, TN)
    q1 = _foldsum(lcyf * lcyf, TN)

    lcx_p = [lcxf[:, k * TN:k * TN + 128] for k in range(_K)]
    lcy_p = [lcyf[:, k * TN:k * TN + 128] for k in range(_K)]
    lcz_p = [lczf[:, k * TN:k * TN + 128] for k in range(_K)]
    if has_g:
        g0_p = [g0f[:, k * TN:k * TN + 128] for k in range(_K)]
        g1_p = [g1f[:, k * TN:k * TN + 128] for k in range(_K)]
        g2_p = [g2f[:, k * TN:k * TN + 128] for k in range(_K)]

    v0 = (q0 - s0 * s0 * (1.0 / _K)) * (1.0 / (_K - 1))
    v1 = (q1 - s1 * s1 * (1.0 / _K)) * (1.0 / (_K - 1))
    tot = jnp.sum(jnp.sqrt(jnp.maximum(v0, 0.0)) + jnp.sqrt(jnp.maximum(v1, 0.0)),
                  keepdims=True)

    z5 = jnp.zeros((5, _K * 128), jnp.float32)
    lcf = jnp.concatenate(
        [jnp.concatenate(lcx_p, axis=1),
         jnp.concatenate(lcy_p, axis=1),
         jnp.concatenate(lcz_p, axis=1), z5], axis=0)

    @pl.when(t == 0)
    def _():
        o_lc[j] = lcf

    if has_g:
        gf = jnp.concatenate(
            [jnp.concatenate(g0_p, axis=1),
             jnp.concatenate(g1_p, axis=1),
             jnp.concatenate(g2_p, axis=1), z5], axis=0)

        @pl.when(t == 0)
        def _():
            o_g[j] = gf

    return tot


def _const_spec(shape):
    n = len(shape)
    return pl.BlockSpec(shape, lambda b, t, _n=n: (0,) * _n)


def _run_axis_level(L, B, tblp, tblc, nbf, dif, aw):
    TN = _TN[L]
    NB = _NB[L]
    pn = _PN[L]
    nt = pn // TN
    has_g = (L == 0)

    out_shapes = [jax.ShapeDtypeStruct((1, 1), jnp.float32),
                  jax.ShapeDtypeStruct((B, 8, _K * 128), jnp.float32)]
    out_specs = [pl.BlockSpec((1, 1), lambda b, t: (0, 0)),
                 pl.BlockSpec((NB, 8, _K * 128), lambda b, t: (b, 0, 0))]
    if has_g:
        out_shapes.append(jax.ShapeDtypeStruct((B, 8, _K * 128), jnp.float32))
        out_specs.append(pl.BlockSpec((NB, 8, _K * 128), lambda b, t: (b, 0, 0)))

    in_specs = [pl.BlockSpec((NB, 8, 128), lambda b, t: (b, 0, 0)),
                pl.BlockSpec((NB, 8, 128), lambda b, t: (b, 0, 0)),
                pl.BlockSpec((NB, 1, 1, _K * TN), lambda b, t: (b, t, 0, 0)),
                pl.BlockSpec((NB, 1, TN), lambda b, t: (b, 0, t))]
    in_specs += [_const_spec(w.shape) for w in aw]

    fn = pl.pallas_call(
        functools.partial(_axis_body, TN, NB, has_g),
        grid=(B // NB, nt),
        in_specs=in_specs,
        out_specs=out_specs,
        out_shape=out_shapes,
        interpret=_INTERPRET,
    )
    return fn(tblp, tblc, nbf, dif, *aw)


def _head_body(lc0, lc1, lc2, lc3, g0r, G0r, xyzr, x3r, nbr,
               w0a, w0b, b0, w02a, w02b, b02,
               w1a, w1b, b1, w12a, w12b, b12,
               w2a, w2b, b2,
               wm1a, wm1b, bm1, wm2, bm2,
               wf1, bf1, wf2, bf2, wf3, bf3,
               o_ref):
    for j in range(_NBH):
        _head_one(j, lc0, lc1, lc2, lc3, g0r, G0r, xyzr, x3r, nbr,
                  w0a, w0b, b0, w02a, w02b, b02,
                  w1a, w1b, b1, w12a, w12b, b12,
                  w2a, w2b, b2,
                  wm1a, wm1b, bm1, wm2, bm2,
                  wf1, bf1, wf2, bf2, wf3, bf3,
                  o_ref)


def _head_one(j, lc0, lc1, lc2, lc3, g0r, G0r, xyzr, x3r, nbr,
              w0a, w0b, b0, w02a, w02b, b02,
              w1a, w1b, b1, w12a, w12b, b12,
              w2a, w2b, b2,
              wm1a, wm1b, bm1, wm2, bm2,
              wf1, bf1, wf2, bf2, wf3, bf3,
              o_ref):
    N = _K * 128
    l0 = lc0[j]
    l1 = lc1[j]
    l2 = lc2[j]
    l3 = lc3[j]
    g0 = g0r[j]
    G0 = G0r[j]
    xyzp = xyzr[j]
    xyz3 = x3r[j]
    nb = nbr[j]

    oh0 = _onehotb(nb[0:1], N)
    grp = _dot(_bf(xyzp), oh0)
    corr = _dot(w0b[...], _bf(G0))
    corr = jnp.concatenate([corr] * _K, axis=1)
    h = _relu(_dot(w0a[...], _bf(l0)) + _dot(w0b[...], _bf(grp)) - corr
              + b0[...])
    P = _maxk(h, 128)

    h = _relu(_dot(w02a[...], _bf(l0)) + _dot(w02b[...], _bf(_dot(_bf(P), oh0)))
              + b02[...])
    P = _maxk(h, 128)

    gat = _dot(_bf(P), _onehotb(nb[1:2], N))
    h = _relu(_dot(w1a[...], _bf(l1)) + _dot(w1b[...], _bf(gat)) + b1[...])
    P = _maxk(h, 128)

    gat = _dot(_bf(P), _onehotb(nb[2:3], N))
    h = _relu(_dot(w12a[...], _bf(l2)) + _dot(w12b[...], _bf(gat)) + b12[...])
    P = _maxk(h, 128)

    gat = _dot(_bf(P), _onehotb(nb[3:4], N))
    h = _relu(_dot(w2a[...], _bf(l3)) + _dot(w2b[...], _bf(gat)) + b2[...])
    P = _maxk(h, 128)

    h = _relu(_dot(wm1a[...], _bf(xyz3)) + _dot(wm1b[...], _bf(P)) + bm1[...])
    h = _relu(_dot(wm2[...], _bf(h)) + bm2[...])
    v = jnp.max(h, axis=1, keepdims=True)

    x = _relu(_dot(wf1[...], v) + bf1[...])
    x = _relu(_dot(wf2[...], x) + bf2[...])
    z = _dot(wf3[...], x) + bf3[...]
    mz = jnp.max(z, axis=0, keepdims=True)
    e = jnp.exp(z - mz)
    se = jnp.sum(e, axis=0, keepdims=True)
    o_ref[j] = z - mz - jnp.log(se)


def _run_head(B, lcs, g0, G0, xyzp, xyz3p, nbf128, hw):
    data = [lcs[0], lcs[1], lcs[2], lcs[3], g0, G0, xyzp, xyz3p, nbf128]
    NB = _NBH
    in_specs = [pl.BlockSpec((NB, 8, _K * 128), lambda b: (b, 0, 0))] * 5
    in_specs += [pl.BlockSpec((NB, 8, 128), lambda b: (b, 0, 0))] * 3
    in_specs += [pl.BlockSpec((NB, 4, _K * 128), lambda b: (b, 0, 0))]
    in_specs += [pl.BlockSpec(w.shape, lambda b, _n=len(w.shape): (0,) * _n)
                 for w in hw]
    fn = pl.pallas_call(
        _head_body,
        grid=(B // NB,),
        in_specs=in_specs,
        out_specs=pl.BlockSpec((NB, 40, 1), lambda b: (b, 0, 0)),
        out_shape=jax.ShapeDtypeStruct((B, 40, 1), jnp.float32),
        interpret=_INTERPRET,
    )
    return fn(*data, *hw)


def _prep_axis_weights(ap):
    s = _S
    (W1, c1), (W2, c2), (W3, c3) = ap['sa1']
    (W4, c4), (W5, c5), (W6, c6) = ap['sa2']
    (W7, c7), (W8, c8), (W9, c9) = ap['sa3']
    f1, f1b = ap['fc1']
    f2, f2b = ap['fc2']
    f3, f3b = ap['fc3']
    bb = lambda v, r: _pad2((v * s).reshape(-1, 1), r, 1)
    cw = lambda w: _bf(jnp.asarray(w, jnp.float32))
    out = [
        cw(_pad2(W1 * s, 8, 8)), bb(c1, 8),
        cw(_pad2(W2 * s, 16, 8)), bb(c2, 16),
        cw(W3 * s), bb(c3, 16),
        cw(_pad2(W4[:, :3] * s, 16, 8)), cw(W4[:, 3:] * s), bb(c4, 16),
        cw(W5 * s), bb(c5, 16),
        cw(W6 * s), bb(c6, 32),
        cw(_pad2(W7[:, :3] * s, 32, 8)), cw(W7[:, 3:] * s), bb(c7, 32),
        cw(W8 * s), bb(c8, 32),
        cw(W9 * s), bb(c9, 64),
        f1.T * s, bb(f1b, 32),
        f2.T * s, bb(f2b, 32),
        _pad2(f3.T, 8, 32), _pad2(f3b.reshape(-1, 1), 8, 1),
    ]
    return [jnp.asarray(w) for w in out]


def _prep_head_weights(p):
    s = _S

    def split(lin, lc_ch, r):
        W, b = lin
        Wt = W.T * s
        return [_bf(_pad2(Wt[:, :lc_ch], r, 8)), _bf(Wt[:, lc_ch:]),
                _pad2((b * s).reshape(-1, 1), r, 1)]

    out = []
    out += split(p['sa0'], 3, 32)
    out[1] = _bf(_pad2(out[1].astype(jnp.float32), 32, 8))
    out += split(p['sa02'], 3, 32)
    out += split(p['sa1'], 3, 128)
    out += split(p['sa12'], 3, 128)
    out += split(p['sa2'], 3, 256)
    m1, m2 = p['merge']
    out += split((m1[0], m1[1]), 3, 256)
    out += [_bf(m2[0].T * s), _pad2((m2[1] * s).reshape(-1, 1), 512, 1)]
    f1, f2, f3 = p['fc1'], p['fc2'], p['fc3']
    out += [f1[0].T * s, _pad2((f1[1] * s).reshape(-1, 1), 256, 1)]
    out += [f2[0].T * s, _pad2((f2[1] * s).reshape(-1, 1), 128, 1)]
    out += [f3[0].T, _pad2(f3[1].reshape(-1, 1), 40, 1)]
    return [jnp.asarray(w) for w in out]


def kernel(xyz, neighbors, data_idxes, params):
    B = xyz.shape[0]
    aw = _prep_axis_weights(params['axis'])
    hw = _prep_head_weights(params)

    xyzp = jnp.zeros((B, 8, 128), jnp.float32).at[:, :3, :].set(
        jnp.transpose(xyz[:, :128, :], (0, 2, 1)))

    nbf, dif = [], []
    for L in range(4):
        pn, TN, cid = _PN[L], _TN[L], _CID[L]
        nt = pn // TN
        nb = neighbors[:, cid:cid + pn, :]
        nbf.append(nb.transpose(0, 2, 1).reshape(B, _K, nt, TN)
                   .transpose(0, 2, 1, 3).reshape(B, nt, 1, _K * TN))
        dif.append(data_idxes[:, cid:cid + pn].reshape(B, 1, pn))

    di128 = jnp.stack([data_idxes[:, _CID[L]:_CID[L] + 128] for L in range(4)],
                      axis=1)
    nbf128 = jnp.stack(
        [neighbors[:, _CID[L]:_CID[L] + 128, :].transpose(0, 2, 1)
         .reshape(B, _K * 128) for L in range(4)], axis=1)

    xyzr = jnp.concatenate(
        [xyz[:, :128, :], jnp.zeros((B, 128, 125), jnp.float32)],
        axis=2).reshape(B * 128, 128)
    tbls, xyz3p = _sc_tables(B, xyzr, di128)

    stds, lcs = [], []
    g0 = None
    prev = xyzp
    for L in range(4):
        outs = _run_axis_level(L, B, prev, tbls[L], nbf[L], dif[L], aw)
        stds.append(outs[0])
        lcs.append(outs[1])
        if L == 0:
            g0 = outs[2]
        prev = tbls[L]

    lc_std = sum(stds[L][0, 0] / (B * _PN[L]) for L in range(4))
    out3 = _run_head(B, lcs, g0, tbls[0], xyzp, xyz3p, nbf128, hw)
    return out3[:, :, 0], jnp.float32(lc_std)


# speedup vs baseline: 1.2644x; 1.0037x over previous
import functools

import jax
import jax.numpy as jnp
from jax import lax
from jax.experimental import pallas as pl
from jax.experimental.pallas import tpu as pltpu
from jax.experimental.pallas import tpu_sc as plsc

_S = float(1.0 / (1.0 + 1e-5) ** 0.5)
_PN = (2048, 512, 512, 128)
_CID = (0, 2048, 2560, 3072)
_K = 32
_TN = (512, 512, 512, 128)
_NB = (2, 2, 2, 4)
_NBH = 2
_INTERPRET = False


def _pad2(a, r, c):
    out = jnp.zeros((r, c), a.dtype)
    return out.at[: a.shape[0], : a.shape[1]].set(a)


def _dot(a, b):
    return jax.lax.dot(a, b, preferred_element_type=jnp.float32)


def _relu(x):
    return jnp.maximum(x, 0.0)


def _bf(x):
    return x.astype(jnp.bfloat16)


def _onehot(idx_row, n):
    io = jax.lax.broadcasted_iota(jnp.int32, (128, idx_row.shape[1]), 0)
    return (io == idx_row).astype(jnp.float32)


def _onehotb(idx_row, n):
    io = jax.lax.broadcasted_iota(jnp.int32, (128, idx_row.shape[1]), 0)
    return (io == idx_row).astype(jnp.bfloat16)


def _hilo(tbl):
    hi = _bf(tbl)
    lo = _bf(tbl - hi.astype(jnp.float32))
    return jnp.concatenate([hi, lo], axis=0)


def _maxk(h, tn):
    w = h.shape[1]
    while w > tn:
        w //= 2
        h = jnp.maximum(h[:, :w], h[:, w:2 * w])
    return h


def _foldsum(v, tn):
    w = v.shape[1]
    while w > tn:
        w //= 2
        v = v[:, :w] + v[:, w:2 * w]
    return v


def _tilek(row):
    return jnp.concatenate([row] * _K, axis=1)


def _sc_level(B, srcg, srch, di, h_from_g):
    mesh = plsc.VectorSubcoreMesh(core_axis_name="c", subcore_axis_name="s")

    @functools.partial(
        pl.kernel, mesh=mesh,
        out_type=[jax.ShapeDtypeStruct((B * 128, 128), jnp.float32),
                  jax.ShapeDtypeStruct((B * 128, 128), jnp.float32)],
        scratch_types=[pltpu.VMEM((128,), jnp.int32),
                       pltpu.VMEM((128,), jnp.int32),
                       pltpu.VMEM((128, 128), jnp.float32),
                       pltpu.VMEM((128, 128), jnp.float32),
                       pltpu.SemaphoreType.DMA,
                       pltpu.SemaphoreType.DMA],
    )
    def k(srcg_h, srch_h, di_h, outg, outh, dvi, idxb, rows, rows2, s1, s2):
        wid = lax.axis_index("s") * 2 + lax.axis_index("c")
        base = wid * 128
        pltpu.sync_copy(di_h.at[wid], dvi)
        for i in range(8):
            idxb[pl.ds(i * 16, 16)] = dvi[pl.ds(i * 16, 16)] + base
        pltpu.async_copy(srcg_h.at[idxb], rows, s1).wait()
        pltpu.sync_copy(rows, outg.at[pl.ds(base, 128)])
        hsrc = outg if h_from_g else srch_h
        pltpu.async_copy(hsrc.at[idxb], rows2, s2).wait()
        pltpu.sync_copy(rows2, outh.at[pl.ds(base, 128)])

    return k(srcg, srch, di)


def _sc_tables(B, xyzr, di128):
    g, h = xyzr, xyzr
    tabs = []
    for L in range(4):
        g, h = _sc_level(B, g, h, di128[:, L], h_from_g=(L == 0))
        tabs.append(g)
    z5 = jnp.zeros((B, 5, 128), jnp.float32)

    def _cm(tb):
        t = jnp.transpose(tb.reshape(B, 128, 128)[:, :, :3], (0, 2, 1))
        return jnp.concatenate([t, z5], axis=1)

    return [_cm(t) for t in tabs], _cm(h)


def _axis_body(TN, NB, has_g,
               tbl_ref, tblc_ref, nbf_ref, dif_ref,
               w1, b1, w2, b2, w3, b3,
               w4a, w4b, b4, w5, b5, w6, b6,
               w7a, w7b, b7, w8, b8, w9, b9,
               f1w, f1b, f2w, f2b, f3w, f3b,
               *outs):
    o_std = outs[0]
    o_lc = outs[1]
    o_g = outs[2] if has_g else None

    b = pl.program_id(0)
    t = pl.program_id(1)

    tots = []
    for j in range(NB):
        tots.append(_axis_one(TN, has_g, j, t,
                              tbl_ref, tblc_ref, nbf_ref, dif_ref,
                              w1, b1, w2, b2, w3, b3,
                              w4a, w4b, b4, w5, b5, w6, b6,
                              w7a, w7b, b7, w8, b8, w9, b9,
                              f1w, f1b, f2w, f2b, f3w, f3b,
                              o_lc, o_g))
    tot = tots[0]
    for v in tots[1:]:
        tot = tot + v

    first = jnp.logical_and(b == 0, t == 0)

    @pl.when(first)
    def _():
        o_std[...] = tot

    @pl.when(jnp.logical_not(first))
    def _():
        o_std[...] = o_std[...] + tot


def _axis_one(TN, has_g, j, t,
              tbl_ref, tblc_ref, nbf_ref, dif_ref,
              w1, b1, w2, b2, w3, b3,
              w4a, w4b, b4, w5, b5, w6, b6,
              w7a, w7b, b7, w8, b8, w9, b9,
              f1w, f1b, f2w, f2b, f3w, f3b,
              o_lc, o_g):
    NKT = _K * TN
    tblp = tbl_ref[j]
    tblc = tblc_ref[j]
    nbf = nbf_ref[j, 0]
    dif = dif_ref[j]

    chl = _dot(_hilo(tblp), _onehotb(dif, TN))
    cur = chl[0:8] + chl[8:16]

    xhl = _dot(_hilo(tblc), _onehotb(nbf, NKT))
    x0 = xhl[0:8] + xhl[8:16]

    x0b = _bf(xhl[0:8])
    h = _relu(_dot(w1[...], x0b) + b1[...])
    h = _relu(_dot(w2[...], _bf(h)) + b2[...])
    l1 = _relu(_dot(w3[...], _bf(h)) + b3[...])
    h = _relu(_dot(w4a[...], x0b) + _dot(w4b[...], _bf(l1)) + b4[...])
    h = _relu(_dot(w5[...], _bf(h)) + b5[...])
    l2 = _relu(_dot(w6[...], _bf(h)) + b6[...])
    h = _relu(_dot(w7a[...], x0b) + _dot(w7b[...], _bf(l2)) + b7[...])
    h = _relu(_dot(w8[...], _bf(h)) + b8[...])
    l3 = _relu(_dot(w9[...], _bf(h)) + b9[...])

    m = _maxk(l3, TN)
    xm = _relu(_dot(f1w[...], m) + f1b[...])
    xm = _relu(_dot(f2w[...], xm) + f2b[...])
    al = _dot(f3w[...], xm) + f3b[...]

    a10, a11, a12 = al[0:1], al[1:2], al[2:3]
    a20, a21, a22 = al[3:4], al[4:5], al[5:6]
    a1n = jnp.sqrt(a10 * a10 + a11 * a11 + a12 * a12) + 1e-9
    kk = (a10 * a20 + a11 * a21 + a12 * a22) / (a1n * a1n)
    b20 = a20 - kk * a10
    b21 = a21 - kk * a11
    b22 = a22 - kk * a12
    bn = jnp.sqrt(b20 * b20 + b21 * b21 + b22 * b22) + 1e-9
    ax0, ax1, ax2 = b20 / bn, b21 / bn, b22 / bn
    az0, az1, az2 = a10 / a1n, a11 / a1n, a12 / a1n
    ay0 = az1 * ax2 - az2 * ax1
    ay1 = az2 * ax0 - az0 * ax2
    ay2 = az0 * ax1 - az1 * ax0

    g0f = x0[0:1] - _tilek(cur[0:1])
    g1f = x0[1:2] - _tilek(cur[1:2])
    g2f = x0[2:3] - _tilek(cur[2:3])
    lcxf = g0f * _tilek(ax0) + g1f * _tilek(ax1) + g2f * _tilek(ax2)
    lcyf = g0f * _tilek(ay0) + g1f * _tilek(ay1) + g2f * _tilek(ay2)
    lczf = g0f * _tilek(az0) + g1f * _tilek(az1) + g2f * _tilek(az2)

    s0 = _foldsum(lcxf, TN)
    q0 = _foldsum(lcxf * lcxf, TN)
    s1 = _foldsum(lcyf, TN)
    q1 = _foldsum(lcyf * lcyf, TN)

    lcx_p = [lcxf[:, k * TN:k * TN + 128] for k in range(_K)]
    lcy_p = [lcyf[:, k * TN:k * TN + 128] for k in range(_K)]
    lcz_p = [lczf[:, k * TN:k * TN + 128] for k in range(_K)]
    if has_g:
        g0_p = [g0f[:, k * TN:k * TN + 128] for k in range(_K)]
        g1_p = [g1f[:, k * TN:k * TN + 128] for k in range(_K)]
        g2_p = [g2f[:, k * TN:k * TN + 128] for k in range(_K)]

    v0 = (q0 - s0 * s0 * (1.0 / _K)) * (1.0 / (_K - 1))
    v1 = (q1 - s1 * s1 * (1.0 / _K)) * (1.0 / (_K - 1))
    tot = jnp.sum(jnp.sqrt(jnp.maximum(v0, 0.0)) + jnp.sqrt(jnp.maximum(v1, 0.0)),
                  keepdims=True)

    z5 = jnp.zeros((5, _K * 128), jnp.float32)
    lcf = jnp.concatenate(
        [jnp.concatenate(lcx_p, axis=1),
         jnp.concatenate(lcy_p, axis=1),
         jnp.concatenate(lcz_p, axis=1), z5], axis=0)

    @pl.when(t == 0)
    def _():
        o_lc[j] = lcf

    if has_g:
        gf = jnp.concatenate(
            [jnp.concatenate(g0_p, axis=1),
             jnp.concatenate(g1_p, axis=1),
             jnp.concatenate(g2_p, axis=1), z5], axis=0)

        @pl.when(t == 0)
        def _():
            o_g[j] = gf

    return tot


def _const_spec(shape):
    n = len(shape)
    return pl.BlockSpec(shape, lambda b, t, _n=n: (0,) * _n)


def _run_axis_level(L, B, tblp, tblc, nbf, dif, aw):
    TN = _TN[L]
    NB = _NB[L]
    pn = _PN[L]
    nt = pn // TN
    has_g = (L == 0)

    out_shapes = [jax.ShapeDtypeStruct((1, 1), jnp.float32),
                  jax.ShapeDtypeStruct((B, 8, _K * 128), jnp.float32)]
    out_specs = [pl.BlockSpec((1, 1), lambda b, t: (0, 0)),
                 pl.BlockSpec((NB, 8, _K * 128), lambda b, t: (b, 0, 0))]
    if has_g:
        out_shapes.append(jax.ShapeDtypeStruct((B, 8, _K * 128), jnp.float32))
        out_specs.append(pl.BlockSpec((NB, 8, _K * 128), lambda b, t: (b, 0, 0)))

    in_specs = [pl.BlockSpec((NB, 8, 128), lambda b, t: (b, 0, 0)),
                pl.BlockSpec((NB, 8, 128), lambda b, t: (b, 0, 0)),
                pl.BlockSpec((NB, 1, 1, _K * TN), lambda b, t: (b, t, 0, 0)),
                pl.BlockSpec((NB, 1, TN), lambda b, t: (b, 0, t))]
    in_specs += [_const_spec(w.shape) for w in aw]

    fn = pl.pallas_call(
        functools.partial(_axis_body, TN, NB, has_g),
        grid=(B // NB, nt),
        in_specs=in_specs,
        out_specs=out_specs,
        out_shape=out_shapes,
        interpret=_INTERPRET,
    )
    return fn(tblp, tblc, nbf, dif, *aw)


def _head_body(lc0, lc1, lc2, lc3, g0r, G0r, xyzr, x3r, nbr,
               w0a, w0b, b0, w02a, w02b, b02,
               w1a, w1b, b1, w12a, w12b, b12,
               w2a, w2b, b2,
               wm1a, wm1b, bm1, wm2, bm2,
               wf1, bf1, wf2, bf2, wf3, bf3,
               o_ref):
    for j in range(_NBH):
        _head_one(j, lc0, lc1, lc2, lc3, g0r, G0r, xyzr, x3r, nbr,
                  w0a, w0b, b0, w02a, w02b, b02,
                  w1a, w1b, b1, w12a, w12b, b12,
                  w2a, w2b, b2,
                  wm1a, wm1b, bm1, wm2, bm2,
                  wf1, bf1, wf2, bf2, wf3, bf3,
                  o_ref)


def _head_one(j, lc0, lc1, lc2, lc3, g0r, G0r, xyzr, x3r, nbr,
              w0a, w0b, b0, w02a, w02b, b02,
              w1a, w1b, b1, w12a, w12b, b12,
              w2a, w2b, b2,
              wm1a, wm1b, bm1, wm2, bm2,
              wf1, bf1, wf2, bf2, wf3, bf3,
              o_ref):
    N = _K * 128
    l0 = lc0[j]
    l1 = lc1[j]
    l2 = lc2[j]
    l3 = lc3[j]
    g0 = g0r[j]
    G0 = G0r[j]
    xyzp = xyzr[j]
    xyz3 = x3r[j]
    nb = nbr[j]

    oh0 = _onehotb(nb[0:1], N)
    grp = _dot(_bf(xyzp), oh0)
    corr = _dot(w0b[...], _bf(G0))
    corr = jnp.concatenate([corr] * _K, axis=1)
    h = _relu(_dot(w0a[...], _bf(l0)) + _dot(w0b[...], _bf(grp)) - corr
              + b0[...])
    P = _maxk(h, 128)

    h = _relu(_dot(w02a[...], _bf(l0)) + _dot(w02b[...], _bf(_dot(_bf(P), oh0)))
              + b02[...])
    P = _maxk(h, 128)

    gat = _dot(_bf(P), _onehotb(nb[1:2], N))
    h = _relu(_dot(w1a[...], _bf(l1)) + _dot(w1b[...], _bf(gat)) + b1[...])
    P = _maxk(h, 128)

    gat = _dot(_bf(P), _onehotb(nb[2:3], N))
    h = _relu(_dot(w12a[...], _bf(l2)) + _dot(w12b[...], _bf(gat)) + b12[...])
    P = _maxk(h, 128)

    gat = _dot(_bf(P), _onehotb(nb[3:4], N))
    h = _relu(_dot(w2a[...], _bf(l3)) + _dot(w2b[...], _bf(gat)) + b2[...])
    P = _maxk(h, 128)

    h = _relu(_dot(wm1a[...], _bf(xyz3)) + _dot(wm1b[...], _bf(P)) + bm1[...])
    h = _relu(_dot(wm2[...], _bf(h)) + bm2[...])
    v = jnp.max(h, axis=1, keepdims=True)

    x = _relu(_dot(wf1[...], v) + bf1[...])
    x = _relu(_dot(wf2[...], x) + bf2[...])
    z = _dot(wf3[...], x) + bf3[...]
    mz = jnp.max(z, axis=0, keepdims=True)
    e = jnp.exp(z - mz)
    se = jnp.sum(e, axis=0, keepdims=True)
    o_ref[j] = z - mz - jnp.log(se)


def _run_head(B, lcs, g0, G0, xyzp, xyz3p, nbf128, hw):
    data = [lcs[0], lcs[1], lcs[2], lcs[3], g0, G0, xyzp, xyz3p, nbf128]
    NB = _NBH
    in_specs = [pl.BlockSpec((NB, 8, _K * 128), lambda b: (b, 0, 0))] * 5
    in_specs += [pl.BlockSpec((NB, 8, 128), lambda b: (b, 0, 0))] * 3
    in_specs += [pl.BlockSpec((NB, 4, _K * 128), lambda b: (b, 0, 0))]
    in_specs += [pl.BlockSpec(w.shape, lambda b, _n=len(w.shape): (0,) * _n)
                 for w in hw]
    fn = pl.pallas_call(
        _head_body,
        grid=(B // NB,),
        in_specs=in_specs,
        out_specs=pl.BlockSpec((NB, 40, 1), lambda b: (b, 0, 0)),
        out_shape=jax.ShapeDtypeStruct((B, 40, 1), jnp.float32),
        interpret=_INTERPRET,
    )
    return fn(*data, *hw)


def _prep_axis_weights(ap):
    s = _S
    (W1, c1), (W2, c2), (W3, c3) = ap['sa1']
    (W4, c4), (W5, c5), (W6, c6) = ap['sa2']
    (W7, c7), (W8, c8), (W9, c9) = ap['sa3']
    f1, f1b = ap['fc1']
    f2, f2b = ap['fc2']
    f3, f3b = ap['fc3']
    bb = lambda v, r: _pad2((v * s).reshape(-1, 1), r, 1)
    cw = lambda w: _bf(jnp.asarray(w, jnp.float32))
    out = [
        cw(_pad2(W1 * s, 8, 8)), bb(c1, 8),
        cw(_pad2(W2 * s, 16, 8)), bb(c2, 16),
        cw(W3 * s), bb(c3, 16),
        cw(_pad2(W4[:, :3] * s, 16, 8)), cw(W4[:, 3:] * s), bb(c4, 16),
        cw(W5 * s), bb(c5, 16),
        cw(W6 * s), bb(c6, 32),
        cw(_pad2(W7[:, :3] * s, 32, 8)), cw(W7[:, 3:] * s), bb(c7, 32),
        cw(W8 * s), bb(c8, 32),
        cw(W9 * s), bb(c9, 64),
        f1.T * s, bb(f1b, 32),
        f2.T * s, bb(f2b, 32),
        _pad2(f3.T, 8, 32), _pad2(f3b.reshape(-1, 1), 8, 1),
    ]
    return [jnp.asarray(w) for w in out]


def _prep_head_weights(p):
    s = _S

    def split(lin, lc_ch, r):
        W, b = lin
        Wt = W.T * s
        return [_bf(_pad2(Wt[:, :lc_ch], r, 8)), _bf(Wt[:, lc_ch:]),
                _pad2((b * s).reshape(-1, 1), r, 1)]

    out = []
    out += split(p['sa0'], 3, 32)
    out[1] = _bf(_pad2(out[1].astype(jnp.float32), 32, 8))
    out += split(p['sa02'], 3, 32)
    out += split(p['sa1'], 3, 128)
    out += split(p['sa12'], 3, 128)
    out += split(p['sa2'], 3, 256)
    m1, m2 = p['merge']
    out += split((m1[0], m1[1]), 3, 256)
    out += [_bf(m2[0].T * s), _pad2((m2[1] * s).reshape(-1, 1), 512, 1)]
    f1, f2, f3 = p['fc1'], p['fc2'], p['fc3']
    out += [f1[0].T * s, _pad2((f1[1] * s).reshape(-1, 1), 256, 1)]
    out += [f2[0].T * s, _pad2((f2[1] * s).reshape(-1, 1), 128, 1)]
    out += [f3[0].T, _pad2(f3[1].reshape(-1, 1), 40, 1)]
    return [jnp.asarray(w) for w in out]


def kernel(xyz, neighbors, data_idxes, params):
    B = xyz.shape[0]
    aw = _prep_axis_weights(params['axis'])
    hw = _prep_head_weights(params)

    xyzp = jnp.zeros((B, 8, 128), jnp.float32).at[:, :3, :].set(
        jnp.transpose(xyz[:, :128, :], (0, 2, 1)))

    nbf, dif = [], []
    for L in range(4):
        pn, TN, cid = _PN[L], _TN[L], _CID[L]
        nt = pn // TN
        nb = neighbors[:, cid:cid + pn, :]
        nbf.append(nb.transpose(0, 2, 1).reshape(B, _K, nt, TN)
                   .transpose(0, 2, 1, 3).reshape(B, nt, 1, _K * TN))
        dif.append(data_idxes[:, cid:cid + pn].reshape(B, 1, pn))

    di128 = jnp.stack([data_idxes[:, _CID[L]:_CID[L] + 128] for L in range(4)],
                      axis=1)
    nbf128 = jnp.stack(
        [neighbors[:, _CID[L]:_CID[L] + 128, :].transpose(0, 2, 1)
         .reshape(B, _K * 128) for L in range(4)], axis=1)

    xyzr = jnp.concatenate(
        [xyz[:, :128, :], jnp.zeros((B, 128, 125), jnp.float32)],
        axis=2).reshape(B * 128, 128)
    tbls, xyz3p = _sc_tables(B, xyzr, di128)

    stds, lcs = [], []
    g0 = None
    prev = xyzp
    for L in range(4):
        outs = _run_axis_level(L, B, prev, tbls[L], nbf[L], dif[L], aw)
        stds.append(outs[0])
        lcs.append(outs[1])
        if L == 0:
            g0 = outs[2]
        prev = tbls[L]

    lc_std = sum(stds[L][0, 0] / (B * _PN[L]) for L in range(4))
    out3 = _run_head(B, lcs, g0, tbls[0], xyzp, xyz3p, nbf128, hw)
    return out3[:, :, 0], jnp.float32(lc_std)
